# baseline (speedup 1.0000x reference)
import functools

import jax
import jax.numpy as jnp
from jax import lax
from jax.experimental import pallas as pl
from jax.experimental.pallas import tpu as pltpu
from jax.experimental.pallas import tpu_sc as plsc

_N = 50
_NP = 64
_F = 5
_E = 800
_L = 16
_CH = _E // _L
_NCH = _NP // _L
_TBL = 1024

_W1_OFF = 0
_W2_OFF = 25
_WL_OFF = 50
_B1_OFF = 55
_B2_OFF = 60
_BL_OFF = 65
_NPAR = 66

_X_OFF = 0
_PAR_OFF = _F * _NP
_TBL_OFF = _PAR_OFF + _NPAR * _L
_STG = _TBL_OFF + _TBL


def _round_bits(i):
    return (i + jnp.int32(0x7FFF) + ((i >> 16) & 1)) & jnp.int32(-65536)


def _bf16_round_sc(v):
    return plsc.bitcast(_round_bits(plsc.bitcast(v, jnp.int32)), jnp.float32)


def _bf16_round_tc(v):
    return lax.bitcast_convert_type(
        _round_bits(lax.bitcast_convert_type(v, jnp.int32)), jnp.float32)


def _sc_body(ei_hbm, stg_hbm, out_hbm,
             src_v, dst_v, stg_v, h_v, xw_v, agg_v, norm_v,
             deg_v, dinv_v, o_v, sem):
    is_t0 = (lax.axis_index("c") == 0) & (lax.axis_index("s") == 0)

    @pl.when(is_t0)
    def _():
        c1 = pltpu.async_copy(ei_hbm.at[0], src_v, sem)
        c2 = pltpu.async_copy(ei_hbm.at[1], dst_v, sem)
        c3 = pltpu.async_copy(stg_hbm, stg_v, sem)
        c1.wait()
        c2.wait()
        c3.wait()
        x_v = stg_v.at[pl.ds(_X_OFF, _F * _NP)]
        par_v = stg_v.at[pl.ds(_PAR_OFF, _NPAR * _L)]
        tbl_v = stg_v.at[pl.ds(_TBL_OFF, _TBL)]

        zeros = jnp.zeros((_L,), jnp.float32)
        ones = jnp.ones((_L,), jnp.float32)

        for i in range(_NCH):
            deg_v[pl.ds(i * _L, _L)] = ones

        @plsc.parallel_loop(0, _E, step=_L)
        def _deg(base):
            d_idx = dst_v[pl.ds(base, _L)]
            plsc.addupdate_scatter(deg_v, [d_idx], ones)

        for i in range(_NCH):
            sl = pl.ds(i * _L, _L)
            di = deg_v[sl].astype(jnp.int32)
            dinv_v[sl] = plsc.load_gather(tbl_v, [di])

        @plsc.parallel_loop(0, _E, step=_L)
        def _norm(base):
            s_idx = src_v[pl.ds(base, _L)]
            d_idx = dst_v[pl.ds(base, _L)]
            norm_v[pl.ds(base, _L)] = (plsc.load_gather(dinv_v, [s_idx])
                                       * plsc.load_gather(dinv_v, [d_idx]))

        def par(r):
            return par_v[pl.ds(r * _L, _L)]

        for layer in range(2):
            w_off = _W1_OFF if layer == 0 else _W2_OFF
            b_off = _B1_OFF if layer == 0 else _B2_OFF
            feat = x_v if layer == 0 else h_v

            for i in range(_NCH):
                if layer == 0:
                    cols = [feat[pl.ds(k * _NP + i * _L, _L)]
                            for k in range(_F)]
                else:
                    cols = [_bf16_round_sc(feat[pl.ds(k * _NP + i * _L, _L)])
                            for k in range(_F)]
                for j in range(_F):
                    acc = zeros
                    for k in range(_F):
                        acc = acc + par(w_off + k * _F + j) * cols[k]
                    xw_v[pl.ds(j * _NP + i * _L, _L)] = acc

            for i in range(_F * _NCH):
                agg_v[pl.ds(i * _L, _L)] = zeros

            @plsc.parallel_loop(0, _E, step=_L, unroll=5)
            def _edges(base):
                s_idx = src_v[pl.ds(base, _L)]
                d_idx = dst_v[pl.ds(base, _L)]
                norm = norm_v[pl.ds(base, _L)]
                for j in range(_F):
                    off = j * _NP
                    vals = plsc.load_gather(xw_v, [s_idx + off]) * norm
                    plsc.addupdate_scatter(agg_v, [d_idx + off], vals)

            for i in range(_NCH):
                dv = dinv_v[pl.ds(i * _L, _L)]
                d2 = dv * dv
                for j in range(_F):
                    sl = pl.ds(j * _NP + i * _L, _L)
                    h_v[sl] = jnp.maximum(
                        agg_v[sl] + d2 * xw_v[sl] + par(b_off + j), 0.0)

        wl = [par(_WL_OFF + k) for k in range(_F)]
        bl = par(_BL_OFF)
        for i in range(_NCH):
            acc = bl
            for k in range(_F):
                hb = _bf16_round_sc(h_v[pl.ds(k * _NP + i * _L, _L)])
                acc = acc + wl[k] * hb
            o_v[pl.ds(i * _L, _L)] = acc

        pltpu.sync_copy(o_v, out_hbm)


@jax.jit
def _run(ei, stg):
    mesh = plsc.VectorSubcoreMesh(core_axis_name="c", subcore_axis_name="s",
                                  num_cores=1)
    f = pl.kernel(
        _sc_body,
        out_type=jax.ShapeDtypeStruct((_NP,), jnp.float32),
        mesh=mesh,
        compiler_params=pltpu.CompilerParams(needs_layout_passes=False),
        scratch_types=[
            pltpu.VMEM((_E,), jnp.int32),
            pltpu.VMEM((_E,), jnp.int32),
            pltpu.VMEM((_STG,), jnp.float32),
            pltpu.VMEM((_F * _NP,), jnp.float32),
            pltpu.VMEM((_F * _NP,), jnp.float32),
            pltpu.VMEM((_F * _NP,), jnp.float32),
            pltpu.VMEM((_E,), jnp.float32),
            pltpu.VMEM((_NP,), jnp.float32),
            pltpu.VMEM((_NP,), jnp.float32),
            pltpu.VMEM((_NP,), jnp.float32),
            pltpu.SemaphoreType.DMA,
        ],
    )
    return f(ei, stg)


def kernel(x, edge_index, W1, b1, W2, b2, Wlin, blin):
    xcm = (jnp.zeros((_F, _NP), jnp.float32)
           .at[:, :_N].set(_bf16_round_tc(x).T).reshape(_F * _NP))

    def splat(v):
        return jnp.broadcast_to(v.reshape(-1, 1), (v.size, _L))

    params = jnp.concatenate(
        [splat(_bf16_round_tc(W1).reshape(-1)),
         splat(_bf16_round_tc(W2).reshape(-1)),
         splat(_bf16_round_tc(Wlin).reshape(-1)),
         splat(b1), splat(b2), splat(blin)], axis=0).reshape(_NPAR * _L)

    ar = jnp.arange(_TBL, dtype=jnp.float32)
    tbl = jnp.where(ar > 0, ar ** -0.5, 0.0)

    stg = jnp.concatenate([xcm, params, tbl])
    out = _run(edge_index.astype(jnp.int32), stg)
    return out[:_N].reshape(_N, 1)


# speedup vs baseline: 1.0485x; 1.0485x over previous
import functools

import jax
import jax.numpy as jnp
from jax import lax
from jax.experimental import pallas as pl
from jax.experimental.pallas import tpu as pltpu
from jax.experimental.pallas import tpu_sc as plsc

_N = 50
_NP = 64
_F = 5
_E = 800
_L = 16
_CH = _E // _L
_NCH = _NP // _L
_TBL = 1024

_W1_OFF = 0
_W2_OFF = 25
_WL_OFF = 50
_B1_OFF = 55
_B2_OFF = 60
_BL_OFF = 65
_NPAR = 66

_X_OFF = 0
_PAR_OFF = _F * _NP
_TBL_OFF = _PAR_OFF + _NPAR * _L
_STG = _TBL_OFF + _TBL


def _round_bits(i):
    return (i + jnp.int32(0x7FFF) + ((i >> 16) & 1)) & jnp.int32(-65536)


def _bf16_round_sc(v):
    return plsc.bitcast(_round_bits(plsc.bitcast(v, jnp.int32)), jnp.float32)


def _bf16_round_tc(v):
    return lax.bitcast_convert_type(
        _round_bits(lax.bitcast_convert_type(v, jnp.int32)), jnp.float32)


def _sc_body(ei_hbm, stg_hbm, out_hbm,
             src_v, dst_v, stg_v, h_v, xw_v, agg_v, norm_v,
             deg_v, dinv_v, o_v, sem):
    is_t0 = (lax.axis_index("c") == 0) & (lax.axis_index("s") == 0)

    @pl.when(is_t0)
    def _():
        c1 = pltpu.async_copy(ei_hbm.at[0], src_v, sem)
        c2 = pltpu.async_copy(ei_hbm.at[1], dst_v, sem)
        c3 = pltpu.async_copy(stg_hbm, stg_v, sem)
        c1.wait()
        c2.wait()
        c3.wait()
        x_v = stg_v.at[pl.ds(_X_OFF, _F * _NP)]
        par_v = stg_v.at[pl.ds(_PAR_OFF, _NPAR * _L)]
        tbl_v = stg_v.at[pl.ds(_TBL_OFF, _TBL)]

        zeros = jnp.zeros((_L,), jnp.float32)
        ones = jnp.ones((_L,), jnp.float32)

        for i in range(_NCH):
            deg_v[pl.ds(i * _L, _L)] = ones

        @plsc.parallel_loop(0, _E, step=_L)
        def _deg(base):
            d_idx = dst_v[pl.ds(base, _L)]
            plsc.addupdate_scatter(deg_v, [d_idx], ones)

        for i in range(_NCH):
            sl = pl.ds(i * _L, _L)
            di = deg_v[sl].astype(jnp.int32)
            dinv_v[sl] = plsc.load_gather(tbl_v, [di])

        @plsc.parallel_loop(0, _E, step=_L)
        def _norm(base):
            s_idx = src_v[pl.ds(base, _L)]
            d_idx = dst_v[pl.ds(base, _L)]
            norm_v[pl.ds(base, _L)] = (plsc.load_gather(dinv_v, [s_idx])
                                       * plsc.load_gather(dinv_v, [d_idx]))

        def par(r):
            return par_v[pl.ds(r * _L, _L)]

        for layer in range(2):
            w_off = _W1_OFF if layer == 0 else _W2_OFF
            b_off = _B1_OFF if layer == 0 else _B2_OFF
            feat = x_v if layer == 0 else h_v

            for i in range(_NCH):
                if layer == 0:
                    cols = [feat[pl.ds(k * _NP + i * _L, _L)]
                            for k in range(_F)]
                else:
                    cols = [_bf16_round_sc(feat[pl.ds(k * _NP + i * _L, _L)])
                            for k in range(_F)]
                for j in range(_F):
                    acc = zeros
                    for k in range(_F):
                        acc = acc + par(w_off + k * _F + j) * cols[k]
                    xw_v[pl.ds(j * _NP + i * _L, _L)] = acc

            for i in range(_F * _NCH):
                agg_v[pl.ds(i * _L, _L)] = zeros

            @plsc.parallel_loop(0, _E, step=_L)
            def _edges(base):
                s_idx = src_v[pl.ds(base, _L)]
                d_idx = dst_v[pl.ds(base, _L)]
                norm = norm_v[pl.ds(base, _L)]
                for j in range(_F):
                    off = j * _NP
                    vals = plsc.load_gather(xw_v, [s_idx + off]) * norm
                    plsc.addupdate_scatter(agg_v, [d_idx + off], vals)

            for i in range(_NCH):
                dv = dinv_v[pl.ds(i * _L, _L)]
                d2 = dv * dv
                for j in range(_F):
                    sl = pl.ds(j * _NP + i * _L, _L)
                    h_v[sl] = jnp.maximum(
                        agg_v[sl] + d2 * xw_v[sl] + par(b_off + j), 0.0)

        wl = [par(_WL_OFF + k) for k in range(_F)]
        bl = par(_BL_OFF)
        for i in range(_NCH):
            acc = bl
            for k in range(_F):
                hb = _bf16_round_sc(h_v[pl.ds(k * _NP + i * _L, _L)])
                acc = acc + wl[k] * hb
            o_v[pl.ds(i * _L, _L)] = acc

        pltpu.sync_copy(o_v.at[pl.ds(0, _N)], out_hbm)


@jax.jit
def _run(ei, stg):
    mesh = plsc.VectorSubcoreMesh(core_axis_name="c", subcore_axis_name="s",
                                  num_cores=1)
    f = pl.kernel(
        _sc_body,
        out_type=jax.ShapeDtypeStruct((_N,), jnp.float32),
        mesh=mesh,
        compiler_params=pltpu.CompilerParams(needs_layout_passes=False),
        scratch_types=[
            pltpu.VMEM((_E,), jnp.int32),
            pltpu.VMEM((_E,), jnp.int32),
            pltpu.VMEM((_STG,), jnp.float32),
            pltpu.VMEM((_F * _NP,), jnp.float32),
            pltpu.VMEM((_F * _NP,), jnp.float32),
            pltpu.VMEM((_F * _NP,), jnp.float32),
            pltpu.VMEM((_E,), jnp.float32),
            pltpu.VMEM((_NP,), jnp.float32),
            pltpu.VMEM((_NP,), jnp.float32),
            pltpu.VMEM((_NP,), jnp.float32),
            pltpu.SemaphoreType.DMA,
        ],
    )
    return f(ei, stg)


def kernel(x, edge_index, W1, b1, W2, b2, Wlin, blin):
    xcm = (jnp.zeros((_F, _NP), jnp.float32)
           .at[:, :_N].set(_bf16_round_tc(x).T).reshape(_F * _NP))

    def splat(v):
        return jnp.broadcast_to(v.reshape(-1, 1), (v.size, _L))

    params = jnp.concatenate(
        [splat(_bf16_round_tc(W1).reshape(-1)),
         splat(_bf16_round_tc(W2).reshape(-1)),
         splat(_bf16_round_tc(Wlin).reshape(-1)),
         splat(b1), splat(b2), splat(blin)], axis=0).reshape(_NPAR * _L)

    ar = jnp.arange(_TBL, dtype=jnp.float32)
    tbl = jnp.where(ar > 0, ar ** -0.5, 0.0)

    stg = jnp.concatenate([xcm, params, tbl])
    out = _run(edge_index.astype(jnp.int32), stg)
    return out.reshape(_N, 1)
